# Initial kernel scaffold; baseline (speedup 1.0000x reference)
import jax
import jax.numpy as jnp
from jax.experimental import pallas as pl


def kernel(inputs, router_w, fc1_w, fc1_b, ln_g, ln_b, fc2_w, fc2_b):
    raise NotImplementedError("write your pallas kernel here")


# speedup vs baseline: 1.2803x; 1.2803x over previous
import functools
import jax
import jax.numpy as jnp
from jax import lax
from jax.experimental import pallas as pl
from jax.experimental.pallas import tpu as pltpu

_B, _S, _DIM = 2, 2048, 1024
_E, _FFN, _TOPK = 16, 2048, 128
_EPS = 1e-5


def _router_body(x_ref, rw_ref, logits_ref, probt_ref):
    x = x_ref[0]
    rw = rw_ref[...]
    logits = lax.dot_general(x, rw, (((1,), (1,)), ((), ())),
                             preferred_element_type=jnp.float32)
    logits_ref[0] = logits
    m = jnp.max(logits, axis=-1, keepdims=True)
    ex = jnp.exp(logits - m)
    probs = ex / jnp.sum(ex, axis=-1, keepdims=True)
    probt_ref[0] = probs.T


def _router(inputs, router_w):
    return pl.pallas_call(
        _router_body,
        grid=(_B,),
        in_specs=[
            pl.BlockSpec((1, _S, _DIM), lambda b: (b, 0, 0)),
            pl.BlockSpec((_E, _DIM), lambda b: (0, 0)),
        ],
        out_specs=[
            pl.BlockSpec((1, _S, _E), lambda b: (b, 0, 0)),
            pl.BlockSpec((1, _E, _S), lambda b: (b, 0, 0)),
        ],
        out_shape=[
            jax.ShapeDtypeStruct((_B, _S, _E), jnp.float32),
            jax.ShapeDtypeStruct((_B, _E, _S), jnp.float32),
        ],
    )(inputs, router_w)


def _ffn_body(xg_ref, w_ref, fc1w_ref, fc1b_ref, lng_ref, lnb_ref,
              fc2w_ref, fc2b_ref, out_ref):
    xg = xg_ref[0]
    h = lax.dot_general(xg, fc1w_ref[0], (((1,), (1,)), ((), ())),
                        preferred_element_type=jnp.float32)
    h = h + fc1b_ref[0]
    h = 0.5 * h * (1.0 + lax.erf(h * 0.7071067811865476))
    mu = jnp.mean(h, axis=-1, keepdims=True)
    var = jnp.mean(jnp.square(h - mu), axis=-1, keepdims=True)
    h = (h - mu) * lax.rsqrt(var + _EPS) * lng_ref[0] + lnb_ref[0]
    y = lax.dot_general(h, fc2w_ref[0], (((1,), (1,)), ((), ())),
                        preferred_element_type=jnp.float32)
    y = y + fc2b_ref[0]
    out_ref[0] = y * w_ref[0, 0][:, None]


def _ffn(xg, wsel, fc1_w, fc1_b, ln_g, ln_b, fc2_w, fc2_b):
    grid = (_B * _E,)
    return pl.pallas_call(
        _ffn_body,
        grid=grid,
        in_specs=[
            pl.BlockSpec((1, _TOPK, _DIM), lambda i: (i, 0, 0)),
            pl.BlockSpec((1, 1, _TOPK), lambda i: (i, 0, 0)),
            pl.BlockSpec((1, _FFN, _DIM), lambda i: (i % _E, 0, 0)),
            pl.BlockSpec((1, 1, _FFN), lambda i: (i % _E, 0, 0)),
            pl.BlockSpec((1, 1, _FFN), lambda i: (i % _E, 0, 0)),
            pl.BlockSpec((1, 1, _FFN), lambda i: (i % _E, 0, 0)),
            pl.BlockSpec((1, _DIM, _FFN), lambda i: (i % _E, 0, 0)),
            pl.BlockSpec((1, 1, _DIM), lambda i: (i % _E, 0, 0)),
        ],
        out_specs=pl.BlockSpec((1, _TOPK, _DIM), lambda i: (i, 0, 0)),
        out_shape=jax.ShapeDtypeStruct((_B * _E, _TOPK, _DIM), jnp.float32),
    )(xg, wsel, fc1_w, fc1_b, ln_g, ln_b, fc2_w, fc2_b)


def kernel(inputs, router_w, fc1_w, fc1_b, ln_g, ln_b, fc2_w, fc2_b):
    router_logits, probt = _router(inputs, router_w)

    weights, selected = lax.top_k(probt, _TOPK)
    xg = jnp.take_along_axis(inputs[:, None, :, :],
                             selected[..., None], axis=2)

    contrib = _ffn(xg.reshape(_B * _E, _TOPK, _DIM),
                   weights.reshape(_B * _E, 1, _TOPK),
                   fc1_w, fc1_b.reshape(_E, 1, _FFN),
                   ln_g.reshape(_E, 1, _FFN), ln_b.reshape(_E, 1, _FFN),
                   fc2_w, fc2_b.reshape(_E, 1, _DIM))
    contrib = contrib.reshape(_B, _E * _TOPK, _DIM)

    flat_idx = selected.reshape(_B, _E * _TOPK)
    out = jnp.zeros((_B, _S, _DIM), jnp.float32)
    out = jax.vmap(lambda o, i, c: o.at[i].add(c))(out, flat_idx, contrib)

    return out, router_logits
